# baseline (speedup 1.0000x reference)
import functools

import jax
import jax.numpy as jnp
from jax import lax
from jax.experimental import pallas as pl
from jax.experimental.pallas import tpu as pltpu
from jax.experimental.pallas import tpu_sc as plsc

SEQ = 128
D = 64
CHUNK = 128


@functools.partial(jax.jit, static_argnames=("b_flat",))
def _sc_embed(x2, embedding, pos_embedding, b_flat):
    info = plsc.get_sparse_core_info()
    nc, ns = info.num_cores, info.num_subcores
    nw = nc * ns
    rows_per_w = b_flat // nw
    n_chunks = rows_per_w // CHUNK

    mesh = plsc.VectorSubcoreMesh(core_axis_name="c", subcore_axis_name="s")

    @functools.partial(
        pl.kernel,
        mesh=mesh,
        compiler_params=pltpu.CompilerParams(use_tc_tiling_on_sc=False),
        out_type=jax.ShapeDtypeStruct((b_flat, D), jnp.float32),
        scratch_types=[
            pltpu.VMEM((n_chunks, CHUNK), jnp.int32),
            pltpu.VMEM((SEQ, D), jnp.float32),
            pltpu.VMEM((CHUNK, D), jnp.float32),
            pltpu.VMEM((CHUNK, D), jnp.float32),
            pltpu.SemaphoreType.DMA,
            pltpu.SemaphoreType.DMA,
        ],
    )
    def k(x_hbm, emb_hbm, pos_hbm, out_hbm, idx_v, pos_v, buf0, buf1,
          sem0, sem1):
        wid = lax.axis_index("s") * nc + lax.axis_index("c")
        base = wid * rows_per_w

        pltpu.sync_copy(x_hbm.at[pl.ds(wid * n_chunks, n_chunks)], idx_v)
        pltpu.sync_copy(pos_hbm, pos_v)

        bufs = ((buf0, sem0), (buf1, sem1))

        def start_gather(c, buf, sem):
            pltpu.make_async_copy(emb_hbm.at[idx_v.at[c]], buf, sem).start()

        start_gather(0, buf0, sem0)
        start_gather(1, buf1, sem1)

        def body(i, carry):
            for b, (buf, sem) in enumerate(bufs):
                c = i * 2 + b
                pltpu.make_async_copy(emb_hbm.at[idx_v.at[c]], buf, sem).wait()

                def add_body(j, acc):
                    for u in range(4):
                        sl = pl.ds(u * 16, 16)
                        plsc.addupdate(buf.at[j, sl], pos_v[j, sl])
                    return acc
                lax.fori_loop(0, CHUNK, add_body, 0, unroll=4)

                pltpu.sync_copy(buf, out_hbm.at[pl.ds(base + c * CHUNK, CHUNK)])
                start_gather(lax.rem(c + 2, n_chunks), buf, sem)
            return carry

        lax.fori_loop(0, n_chunks // 2, body, 0)

        pltpu.make_async_copy(emb_hbm.at[idx_v.at[0]], buf0, sem0).wait()
        pltpu.make_async_copy(emb_hbm.at[idx_v.at[1]], buf1, sem1).wait()

    return k(x2, embedding, pos_embedding)


def kernel(x, embedding, pos_embedding):
    b, s = x.shape
    b_flat = b * s
    x2 = x.reshape(b_flat // CHUNK, CHUNK).astype(jnp.int32)
    out = _sc_embed(x2, embedding, pos_embedding, b_flat)
    return out.reshape(b, s, D)
